# baseline (speedup 1.0000x reference)
import functools

import numpy as np

import jax
import jax.numpy as jnp
from jax import lax
from jax.experimental import pallas as pl
from jax.experimental.pallas import tpu as pltpu
from jax.experimental.pallas import tpu_sc as plsc

H = 128
L = 16
NC = 2
NS = 16
CH = 128

_PERM = np.empty(H, np.int32)
for _k in range(H):
    _j, _t = divmod(_k, 32)
    _PERM[_k] = 32 * _j + (_t // 2 if _t % 2 == 0 else 16 + _t // 2)


def _swish(v):
    return v * jax.nn.sigmoid(v)


def _xlin_body(x_ref, w1_ref, b1_ref, w2_ref, b2_ref, o1_ref, o2_ref):
    xb = x_ref[...]
    o1_ref[...] = _swish(
        jnp.dot(xb, w1_ref[...], preferred_element_type=jnp.float32) + b1_ref[...])
    o2_ref[...] = _swish(
        jnp.dot(xb, w2_ref[...], preferred_element_type=jnp.float32) + b2_ref[...])


def _run_xlin(x, w1t, b1, w2t, b2):
    n = x.shape[0]
    bn = 2000
    grid = n // bn
    full = lambda i: (0, 0)
    return pl.pallas_call(
        _xlin_body,
        grid=(grid,),
        in_specs=[
            pl.BlockSpec((bn, H), lambda i: (i, 0)),
            pl.BlockSpec((H, H), full),
            pl.BlockSpec((1, H), full),
            pl.BlockSpec((H, H), full),
            pl.BlockSpec((1, H), full),
        ],
        out_specs=[
            pl.BlockSpec((bn, H), lambda i: (i, 0)),
            pl.BlockSpec((bn, H), lambda i: (i, 0)),
        ],
        out_shape=[
            jax.ShapeDtypeStruct((n, H), jnp.float32),
            jax.ShapeDtypeStruct((n, H), jnp.float32),
        ],
    )(x, w1t, b1, w2t, b2)


def _edgefeat_body(feat_ref, pos_ref, wa1_ref, wa2_ref, wb1_ref, wb2_ref,
                   f1_ref, f2_ref):
    m1 = jnp.dot(feat_ref[...], wa1_ref[...], preferred_element_type=jnp.float32)
    f1_ref[...] = jnp.dot(
        m1, wa2_ref[...], preferred_element_type=jnp.float32).astype(jnp.bfloat16)
    m2 = jnp.dot(pos_ref[...], wb1_ref[...], preferred_element_type=jnp.float32)
    f2_ref[...] = jnp.dot(
        m2, wb2_ref[...], preferred_element_type=jnp.float32).astype(jnp.bfloat16)


def _run_edgefeat(feat, pos, wa1t, wa2t, wb1t, wb2t):
    e_pad = feat.shape[0]
    f1d = feat.shape[1]
    ped = pos.shape[1]
    mid = wa1t.shape[1]
    be = 4000
    grid = e_pad // be
    full = lambda i: (0, 0)
    return pl.pallas_call(
        _edgefeat_body,
        grid=(grid,),
        in_specs=[
            pl.BlockSpec((be, f1d), lambda i: (i, 0)),
            pl.BlockSpec((be, ped), lambda i: (i, 0)),
            pl.BlockSpec((f1d, mid), full),
            pl.BlockSpec((mid, H), full),
            pl.BlockSpec((ped, mid), full),
            pl.BlockSpec((mid, H), full),
        ],
        out_specs=[
            pl.BlockSpec((be, H), lambda i: (i, 0)),
            pl.BlockSpec((be, H), lambda i: (i, 0)),
        ],
        out_shape=[
            jax.ShapeDtypeStruct((e_pad, H), jnp.bfloat16),
            jax.ShapeDtypeStruct((e_pad, H), jnp.bfloat16),
        ],
    )(feat, pos, wa1t, wa2t, wb1t, wb2t)


def _make_sc_conv(n_nodes, e_total):
    eps = e_total // NS
    nchunk = eps // CH
    rem = eps - nchunk * CH
    rows_per_sub = n_nodes // NS
    zchunks = [CH] * (rows_per_sub // CH)
    if rows_per_sub % CH:
        zchunks.append(rows_per_sub % CH)

    mesh = plsc.VectorSubcoreMesh(core_axis_name="c", subcore_axis_name="s")

    @functools.partial(
        pl.kernel,
        out_type=(
            jax.ShapeDtypeStruct((n_nodes, H), jnp.float32),
            jax.ShapeDtypeStruct((n_nodes, H), jnp.float32),
        ),
        mesh=mesh,
        scratch_types=[
            pltpu.VMEM((2, CH), jnp.int32),
            pltpu.VMEM((2, CH), jnp.int32),
            pltpu.VMEM((rem if rem else 8,), jnp.int32),
            pltpu.VMEM((rem if rem else 8,), jnp.int32),
            pltpu.VMEM((CH, H), jnp.float32),
            pltpu.VMEM((CH, H), jnp.float32),
            pltpu.VMEM((CH // 2, H), jnp.int32),
            pltpu.VMEM((CH // 2, H), jnp.int32),
            pltpu.VMEM_SHARED((n_nodes, H), jnp.float32),
            pltpu.SemaphoreType.DMA,
            pltpu.SemaphoreType.DMA,
            pltpu.SemaphoreType.DMA,
            pltpu.SemaphoreType.DMA,
            pltpu.SemaphoreType.DMA,
            pltpu.SemaphoreType.DMA,
        ],
    )
    def sc_conv(src_hbm, dst_hbm, ei_hbm, xlin_hbm, f1_hbm, f2_hbm,
                agg1_hbm, agg2_hbm,
                idx0, idx1, sidx_r, didx_r,
                xr0, xr1, fv0, fv1, agg_sp,
                gsem0, gsem1, fsem0, fsem1, ssem0, ssem1):
        c = lax.axis_index("c")
        s = lax.axis_index("s")
        idx = (idx0, idx1)
        xr = (xr0, xr1)
        fv = (fv0, fv1)
        gsem = (gsem0, gsem1)
        fsem = (fsem0, fsem1)
        ssem = (ssem0, ssem1)

        def zrow(r, _):
            for j in range(H // L):
                xr0[r, pl.ds(j * L, L)] = jnp.zeros((L,), jnp.float32)
            return 0
        lax.fori_loop(0, CH, zrow, 0)
        row0 = s * rows_per_sub
        zoff = 0
        for zc in zchunks:
            pltpu.sync_copy(xr0.at[pl.ds(0, zc)],
                            agg_sp.at[pl.ds(row0 + zoff, zc)])
            zoff += zc
        plsc.subcore_barrier()

        def run_conv(f_hbm, out_hbm):
            def issue(ci, b, wait_scatter):
                base2 = pl.multiple_of(s * (eps // 2) + ci * (CH // 2), 8)
                if wait_scatter:
                    pltpu.make_async_copy(xr[b], agg_sp.at[idx[b].at[1]],
                                          ssem[b]).wait()
                pltpu.sync_copy(ei_hbm.at[s, :, pl.ds(ci * CH, CH)], idx[b])
                pltpu.async_copy(xlin_hbm.at[idx[b].at[0]], xr[b], gsem[b])
                pltpu.async_copy(f_hbm.at[pl.ds(base2, CH // 2)], fv[b],
                                 fsem[b])

            def process(ci, b):
                base2 = pl.multiple_of(s * (eps // 2) + ci * (CH // 2), 8)
                pltpu.make_async_copy(xlin_hbm.at[idx[b].at[0]], xr[b],
                                      gsem[b]).wait()
                pltpu.make_async_copy(f_hbm.at[pl.ds(base2, CH // 2)],
                                      fv[b], fsem[b]).wait()

                @plsc.parallel_loop(0, CH // 2, 1, unroll=2)
                def _(q):
                    for p in range(2):
                        r = 2 * q + p
                        for j in range(H // 32):
                            w = fv[b][q, pl.ds(64 * p + L * j, L)]
                            fa = lax.bitcast_convert_type(w << 16, jnp.float32)
                            fb = lax.bitcast_convert_type(
                                w & jnp.int32(-65536), jnp.float32)
                            sl0 = pl.ds(32 * j, L)
                            sl1 = pl.ds(32 * j + L, L)
                            xr[b][r, sl0] = fa * xr[b][r, sl0]
                            xr[b][r, sl1] = fb * xr[b][r, sl1]
                pltpu.async_copy(xr[b], agg_sp.at[idx[b].at[1]], ssem[b],
                                 add=True)

            issue(0, 0, False)
            issue(1, 1, False)

            def pair(t, _):
                c0 = 2 * t
                process(c0, 0)
                issue(c0 + 2, 0, True)
                process(c0 + 1, 1)
                issue(c0 + 3, 1, True)
                return 0
            lax.fori_loop(0, nchunk // 2 - 1, pair, 0)
            process(nchunk - 2, 0)
            process(nchunk - 1, 1)
            pltpu.make_async_copy(xr[0], agg_sp.at[idx0.at[1]], ssem0).wait()
            pltpu.make_async_copy(xr[1], agg_sp.at[idx1.at[1]], ssem1).wait()
            if rem:
                base = s * eps + nchunk * CH
                pltpu.sync_copy(src_hbm.at[pl.ds(base, rem)], sidx_r)
                pltpu.sync_copy(dst_hbm.at[pl.ds(base, rem)], didx_r)
                gat = pltpu.async_copy(xlin_hbm.at[sidx_r],
                                       xr0.at[pl.ds(0, rem)], gsem0)
                base2 = pl.multiple_of(
                    s * (eps // 2) + nchunk * (CH // 2), 8)
                pltpu.sync_copy(f_hbm.at[pl.ds(base2, rem // 2)],
                                fv0.at[pl.ds(0, rem // 2)])
                gat.wait()

                @plsc.parallel_loop(0, rem // 2, 1, unroll=2)
                def _(q):
                    for p in range(2):
                        r = 2 * q + p
                        for j in range(H // 32):
                            w = fv0[q, pl.ds(64 * p + L * j, L)]
                            fa = lax.bitcast_convert_type(w << 16, jnp.float32)
                            fb = lax.bitcast_convert_type(
                                w & jnp.int32(-65536), jnp.float32)
                            sl0 = pl.ds(32 * j, L)
                            sl1 = pl.ds(32 * j + L, L)
                            xr0[r, sl0] = fa * xr0[r, sl0]
                            xr0[r, sl1] = fb * xr0[r, sl1]
                pltpu.sync_copy(xr0.at[pl.ds(0, rem)],
                                agg_sp.at[didx_r], add=True)
            plsc.subcore_barrier()
            pltpu.sync_copy(agg_sp.at[pl.ds(row0, rows_per_sub)],
                            out_hbm.at[pl.ds(row0, rows_per_sub)])

        @pl.when(c == 0)
        def _():
            run_conv(f1_hbm, agg1_hbm)

        @pl.when(c == 1)
        def _():
            run_conv(f2_hbm, agg2_hbm)

    return sc_conv


def _tail_body(agg1_ref, agg2_ref, xl1_ref, xl2_ref,
               c1wl_ref, c1bl_ref, c1wr_ref, wl1_ref, bl1_ref,
               c2wl_ref, c2bl_ref, c2wr_ref, wl2_ref, bl2_ref,
               wc0a_ref, wc0b_ref, bc0_ref, wc1_ref, bc1_ref, wc2_ref, bc2_ref,
               wl0_ref, bl0_ref, wll1_ref, bll1_ref, wf_ref, bf_ref,
               out_ref):
    f32 = jnp.float32
    xl1 = xl1_ref[...]
    a1 = (jnp.dot(agg1_ref[...], c1wl_ref[...], preferred_element_type=f32)
          + c1bl_ref[...]
          + jnp.dot(xl1, c1wr_ref[...], preferred_element_type=f32))
    h1 = _swish(jnp.dot(a1, wl1_ref[...], preferred_element_type=f32) + bl1_ref[...])
    a2 = (jnp.dot(agg2_ref[...], c2wl_ref[...], preferred_element_type=f32)
          + c2bl_ref[...]
          + jnp.dot(xl1, c2wr_ref[...], preferred_element_type=f32))
    h2 = _swish(jnp.dot(a2, wl2_ref[...], preferred_element_type=f32) + bl2_ref[...])
    h = _swish(jnp.dot(h1, wc0a_ref[...], preferred_element_type=f32)
               + jnp.dot(h2, wc0b_ref[...], preferred_element_type=f32)
               + bc0_ref[...])
    h = _swish(jnp.dot(h, wc1_ref[...], preferred_element_type=f32) + bc1_ref[...])
    h = _swish(jnp.dot(h, wc2_ref[...], preferred_element_type=f32) + bc2_ref[...])
    h = h + xl2_ref[...]
    h = _swish(jnp.dot(h, wl0_ref[...], preferred_element_type=f32) + bl0_ref[...])
    h = _swish(jnp.dot(h, wll1_ref[...], preferred_element_type=f32) + bll1_ref[...])
    out_ref[...] = jnp.dot(h, wf_ref[...], preferred_element_type=f32) + bf_ref[...]


def _run_tail(agg1, agg2, xl1, xl2, weights):
    n = xl1.shape[0]
    bn = 2000
    grid = n // bn
    full = lambda i: (0, 0)
    row = lambda i: (i, 0)
    wspecs = []
    for w in weights:
        wspecs.append(pl.BlockSpec(w.shape, full))
    return pl.pallas_call(
        _tail_body,
        grid=(grid,),
        in_specs=[pl.BlockSpec((bn, H), row)] * 4 + wspecs,
        out_specs=pl.BlockSpec((bn, H), row),
        out_shape=jax.ShapeDtypeStruct((n, H), jnp.float32),
    )(agg1, agg2, xl1, xl2, *weights)


def kernel(x, feature1, pos_emb, edge_index, batch, params):
    p = params
    n = x.shape[0]
    e = edge_index.shape[1]
    n_pad = ((n + NS * 8 - 1) // (NS * 8)) * (NS * 8)

    src = edge_index[0]
    dst = edge_index[1]
    eps = e // NS
    ei = jnp.stack([src.reshape(NS, eps), dst.reshape(NS, eps)], axis=1)

    r1 = lambda b: b.reshape(1, -1)

    xl1, xl2 = _run_xlin(x, p['W_lin_1'].T, r1(p['b_lin_1']),
                         p['W_lin_2'].T, r1(p['b_lin_2']))

    f1, f2 = _run_edgefeat(feature1, pos_emb,
                           p['Wf1_1'].T, p['Wf1_2'].T[:, _PERM],
                           p['Wf2_1'].T, p['Wf2_2'].T[:, _PERM])
    f1i = lax.bitcast_convert_type(
        f1.reshape(e, H // 2, 2), jnp.int32).reshape(e // 2, H)
    f2i = lax.bitcast_convert_type(
        f2.reshape(e, H // 2, 2), jnp.int32).reshape(e // 2, H)

    sc_conv = _make_sc_conv(n_pad, e)
    agg1, agg2 = sc_conv(src, dst, ei, xl1, f1i, f2i)

    wcat0t = p['Wcat0'].T
    weights = [
        p['conv1_Wl'].T, r1(p['conv1_bl']), p['conv1_Wr'].T,
        p['W_lin1'].T, r1(p['b_lin1']),
        p['conv2_Wl'].T, r1(p['conv2_bl']), p['conv2_Wr'].T,
        p['W_lin2'].T, r1(p['b_lin2']),
        wcat0t[:H], wcat0t[H:], r1(p['bcat0']),
        p['Wcat1'].T, r1(p['bcat1']),
        p['Wcat2'].T, r1(p['bcat2']),
        p['Wl0'].T, r1(p['bl0']),
        p['Wl1'].T, r1(p['bl1']),
        p['Wfinal'].T, r1(p['bfinal']),
    ]
    return _run_tail(agg1, agg2, xl1, xl2, weights)


# speedup vs baseline: 3.2384x; 3.2384x over previous
import functools

import numpy as np

import jax
import jax.numpy as jnp
from jax import lax
from jax.experimental import pallas as pl
from jax.experimental.pallas import tpu as pltpu
from jax.experimental.pallas import tpu_sc as plsc

H = 128
L = 16
NC = 2
NS = 16
CH = 128

BE = 4000

def _edge_perm(e, eps):
    half = BE // 2
    u = np.arange(e // 2)
    elo = (u // half) * BE + (u % half)
    ehi = elo + half
    fps = eps // 2
    out = []
    for s in range(NS):
        base = s * fps
        nfull = fps // 64
        for ci in range(nfull):
            rr = base + 64 * ci + np.arange(64)
            out.append(elo[rr])
            out.append(ehi[rr])
        tail = fps - nfull * 64
        if tail:
            rr = base + nfull * 64 + np.arange(tail)
            out.append(elo[rr])
            out.append(ehi[rr])
    return np.concatenate(out)


def _swish(v):
    return v * jax.nn.sigmoid(v)


def _xlin_body(x_ref, w1_ref, b1_ref, w2_ref, b2_ref, o1_ref, o2_ref):
    xb = x_ref[...]
    o1_ref[...] = _swish(
        jnp.dot(xb, w1_ref[...], preferred_element_type=jnp.float32) + b1_ref[...])
    o2_ref[...] = _swish(
        jnp.dot(xb, w2_ref[...], preferred_element_type=jnp.float32) + b2_ref[...])


def _run_xlin(x, w1t, b1, w2t, b2):
    n = x.shape[0]
    bn = 2000
    grid = n // bn
    full = lambda i: (0, 0)
    return pl.pallas_call(
        _xlin_body,
        grid=(grid,),
        in_specs=[
            pl.BlockSpec((bn, H), lambda i: (i, 0)),
            pl.BlockSpec((H, H), full),
            pl.BlockSpec((1, H), full),
            pl.BlockSpec((H, H), full),
            pl.BlockSpec((1, H), full),
        ],
        out_specs=[
            pl.BlockSpec((bn, H), lambda i: (i, 0)),
            pl.BlockSpec((bn, H), lambda i: (i, 0)),
        ],
        out_shape=[
            jax.ShapeDtypeStruct((n, H), jnp.float32),
            jax.ShapeDtypeStruct((n, H), jnp.float32),
        ],
    )(x, w1t, b1, w2t, b2)


def _edgefeat_body(feat_ref, pos_ref, wa1_ref, wa2_ref, wb1_ref, wb2_ref,
                   f1_ref, f2_ref):
    be = feat_ref.shape[0]

    def pack_bf16_pairs(g):
        u = lax.bitcast_convert_type(g, jnp.uint32)
        rnd = (u + jnp.uint32(0x7FFF) + ((u >> 16) & jnp.uint32(1))) >> 16
        return rnd[:be // 2] | (rnd[be // 2:] << 16)

    m1 = jnp.dot(feat_ref[...], wa1_ref[...], preferred_element_type=jnp.float32)
    g1 = jnp.dot(m1, wa2_ref[...], preferred_element_type=jnp.float32)
    f1_ref[...] = pack_bf16_pairs(g1)
    m2 = jnp.dot(pos_ref[...], wb1_ref[...], preferred_element_type=jnp.float32)
    g2 = jnp.dot(m2, wb2_ref[...], preferred_element_type=jnp.float32)
    f2_ref[...] = pack_bf16_pairs(g2)


def _run_edgefeat(feat, pos, wa1t, wa2t, wb1t, wb2t):
    e_pad = feat.shape[0]
    f1d = feat.shape[1]
    ped = pos.shape[1]
    mid = wa1t.shape[1]
    be = BE
    grid = e_pad // be
    full = lambda i: (0, 0)
    return pl.pallas_call(
        _edgefeat_body,
        grid=(grid,),
        in_specs=[
            pl.BlockSpec((be, f1d), lambda i: (i, 0)),
            pl.BlockSpec((be, ped), lambda i: (i, 0)),
            pl.BlockSpec((f1d, mid), full),
            pl.BlockSpec((mid, H), full),
            pl.BlockSpec((ped, mid), full),
            pl.BlockSpec((mid, H), full),
        ],
        out_specs=[
            pl.BlockSpec((be // 2, H), lambda i: (i, 0)),
            pl.BlockSpec((be // 2, H), lambda i: (i, 0)),
        ],
        out_shape=[
            jax.ShapeDtypeStruct((e_pad // 2, H), jnp.uint32),
            jax.ShapeDtypeStruct((e_pad // 2, H), jnp.uint32),
        ],
    )(feat, pos, wa1t, wa2t, wb1t, wb2t)


def _make_sc_conv(n_nodes, e_total):
    eps = e_total // NS
    nchunk = eps // CH
    rem = eps - nchunk * CH
    rows_per_sub = n_nodes // NS
    zchunks = [CH] * (rows_per_sub // CH)
    if rows_per_sub % CH:
        zchunks.append(rows_per_sub % CH)

    mesh = plsc.VectorSubcoreMesh(core_axis_name="c", subcore_axis_name="s")

    @functools.partial(
        pl.kernel,
        out_type=(
            jax.ShapeDtypeStruct((n_nodes, H), jnp.float32),
            jax.ShapeDtypeStruct((n_nodes, H), jnp.float32),
        ),
        mesh=mesh,
        scratch_types=[
            pltpu.VMEM((2, CH), jnp.int32),
            pltpu.VMEM((2, CH), jnp.int32),
            pltpu.VMEM((rem if rem else 8,), jnp.int32),
            pltpu.VMEM((rem if rem else 8,), jnp.int32),
            pltpu.VMEM((CH, H), jnp.float32),
            pltpu.VMEM((CH, H), jnp.float32),
            pltpu.VMEM((CH // 2, H), jnp.uint32),
            pltpu.VMEM((CH // 2, H), jnp.uint32),
            pltpu.VMEM_SHARED((n_nodes, H), jnp.float32),
            pltpu.SemaphoreType.DMA,
            pltpu.SemaphoreType.DMA,
            pltpu.SemaphoreType.DMA,
            pltpu.SemaphoreType.DMA,
            pltpu.SemaphoreType.DMA,
            pltpu.SemaphoreType.DMA,
        ],
    )
    def sc_conv(src_hbm, dst_hbm, ei_hbm, xlin_hbm, f1_hbm, f2_hbm,
                agg1_hbm, agg2_hbm,
                idx0, idx1, sidx_r, didx_r,
                xr0, xr1, fv0, fv1, agg_sp,
                gsem0, gsem1, fsem0, fsem1, ssem0, ssem1):
        c = lax.axis_index("c")
        s = lax.axis_index("s")
        idx = (idx0, idx1)
        xr = (xr0, xr1)
        fv = (fv0, fv1)
        gsem = (gsem0, gsem1)
        fsem = (fsem0, fsem1)
        ssem = (ssem0, ssem1)

        def zrow(r, _):
            for j in range(H // L):
                xr0[r, pl.ds(j * L, L)] = jnp.zeros((L,), jnp.float32)
            return 0
        lax.fori_loop(0, CH, zrow, 0)
        row0 = s * rows_per_sub
        zoff = 0
        for zc in zchunks:
            pltpu.sync_copy(xr0.at[pl.ds(0, zc)],
                            agg_sp.at[pl.ds(row0 + zoff, zc)])
            zoff += zc
        plsc.subcore_barrier()

        def run_conv(f_hbm, out_hbm):
            def issue(ci, b, wait_scatter):
                base2 = pl.multiple_of(s * (eps // 2) + ci * (CH // 2), 8)
                if wait_scatter:
                    pltpu.make_async_copy(xr[b], agg_sp.at[idx[b].at[1]],
                                          ssem[b]).wait()
                pltpu.sync_copy(ei_hbm.at[s, :, pl.ds(ci * CH, CH)], idx[b])
                pltpu.async_copy(xlin_hbm.at[idx[b].at[0]], xr[b], gsem[b])
                pltpu.async_copy(f_hbm.at[pl.ds(base2, CH // 2)], fv[b],
                                 fsem[b])

            def process(ci, b):
                base2 = pl.multiple_of(s * (eps // 2) + ci * (CH // 2), 8)
                pltpu.make_async_copy(xlin_hbm.at[idx[b].at[0]], xr[b],
                                      gsem[b]).wait()
                pltpu.make_async_copy(f_hbm.at[pl.ds(base2, CH // 2)],
                                      fv[b], fsem[b]).wait()

                @plsc.parallel_loop(0, CH // 2, 1, unroll=2)
                def _(q):
                    for j in range(H // L):
                        sl = pl.ds(L * j, L)
                        w = fv[b][q, sl]
                        fa = lax.bitcast_convert_type(w << 16, jnp.float32)
                        fb = lax.bitcast_convert_type(
                            w & jnp.uint32(0xFFFF0000), jnp.float32)
                        xr[b][q, sl] = fa * xr[b][q, sl]
                        hr = CH // 2 + q
                        xr[b][hr, sl] = fb * xr[b][hr, sl]
                pltpu.async_copy(xr[b], agg_sp.at[idx[b].at[1]], ssem[b],
                                 add=True)

            issue(0, 0, False)
            issue(1, 1, False)

            def pair(t, _):
                c0 = 2 * t
                process(c0, 0)
                issue(c0 + 2, 0, True)
                process(c0 + 1, 1)
                issue(c0 + 3, 1, True)
                return 0
            lax.fori_loop(0, nchunk // 2 - 1, pair, 0)
            process(nchunk - 2, 0)
            process(nchunk - 1, 1)
            pltpu.make_async_copy(xr[0], agg_sp.at[idx0.at[1]], ssem0).wait()
            pltpu.make_async_copy(xr[1], agg_sp.at[idx1.at[1]], ssem1).wait()
            if rem:
                base = s * eps + nchunk * CH
                pltpu.sync_copy(src_hbm.at[pl.ds(base, rem)], sidx_r)
                pltpu.sync_copy(dst_hbm.at[pl.ds(base, rem)], didx_r)
                gat = pltpu.async_copy(xlin_hbm.at[sidx_r],
                                       xr0.at[pl.ds(0, rem)], gsem0)
                base2 = pl.multiple_of(
                    s * (eps // 2) + nchunk * (CH // 2), 8)
                pltpu.sync_copy(f_hbm.at[pl.ds(base2, rem // 2)],
                                fv0.at[pl.ds(0, rem // 2)])
                gat.wait()

                @plsc.parallel_loop(0, rem // 2, 1, unroll=2)
                def _(q):
                    for j in range(H // L):
                        sl = pl.ds(L * j, L)
                        w = fv0[q, sl]
                        fa = lax.bitcast_convert_type(w << 16, jnp.float32)
                        fb = lax.bitcast_convert_type(
                            w & jnp.uint32(0xFFFF0000), jnp.float32)
                        xr0[q, sl] = fa * xr0[q, sl]
                        hr = rem // 2 + q
                        xr0[hr, sl] = fb * xr0[hr, sl]
                pltpu.sync_copy(xr0.at[pl.ds(0, rem)],
                                agg_sp.at[didx_r], add=True)
            plsc.subcore_barrier()
            pltpu.sync_copy(agg_sp.at[pl.ds(row0, rows_per_sub)],
                            out_hbm.at[pl.ds(row0, rows_per_sub)])

        @pl.when(c == 0)
        def _():
            run_conv(f1_hbm, agg1_hbm)

        @pl.when(c == 1)
        def _():
            run_conv(f2_hbm, agg2_hbm)

    return sc_conv


def _tail_body(agg1_ref, agg2_ref, xl1_ref, xl2_ref,
               c1wl_ref, c1bl_ref, c1wr_ref, wl1_ref, bl1_ref,
               c2wl_ref, c2bl_ref, c2wr_ref, wl2_ref, bl2_ref,
               wc0a_ref, wc0b_ref, bc0_ref, wc1_ref, bc1_ref, wc2_ref, bc2_ref,
               wl0_ref, bl0_ref, wll1_ref, bll1_ref, wf_ref, bf_ref,
               out_ref):
    f32 = jnp.float32
    xl1 = xl1_ref[...]
    a1 = (jnp.dot(agg1_ref[...], c1wl_ref[...], preferred_element_type=f32)
          + c1bl_ref[...]
          + jnp.dot(xl1, c1wr_ref[...], preferred_element_type=f32))
    h1 = _swish(jnp.dot(a1, wl1_ref[...], preferred_element_type=f32) + bl1_ref[...])
    a2 = (jnp.dot(agg2_ref[...], c2wl_ref[...], preferred_element_type=f32)
          + c2bl_ref[...]
          + jnp.dot(xl1, c2wr_ref[...], preferred_element_type=f32))
    h2 = _swish(jnp.dot(a2, wl2_ref[...], preferred_element_type=f32) + bl2_ref[...])
    h = _swish(jnp.dot(h1, wc0a_ref[...], preferred_element_type=f32)
               + jnp.dot(h2, wc0b_ref[...], preferred_element_type=f32)
               + bc0_ref[...])
    h = _swish(jnp.dot(h, wc1_ref[...], preferred_element_type=f32) + bc1_ref[...])
    h = _swish(jnp.dot(h, wc2_ref[...], preferred_element_type=f32) + bc2_ref[...])
    h = h + xl2_ref[...]
    h = _swish(jnp.dot(h, wl0_ref[...], preferred_element_type=f32) + bl0_ref[...])
    h = _swish(jnp.dot(h, wll1_ref[...], preferred_element_type=f32) + bll1_ref[...])
    out_ref[...] = jnp.dot(h, wf_ref[...], preferred_element_type=f32) + bf_ref[...]


def _run_tail(agg1, agg2, xl1, xl2, weights):
    n = xl1.shape[0]
    bn = 2000
    grid = n // bn
    full = lambda i: (0, 0)
    row = lambda i: (i, 0)
    wspecs = []
    for w in weights:
        wspecs.append(pl.BlockSpec(w.shape, full))
    return pl.pallas_call(
        _tail_body,
        grid=(grid,),
        in_specs=[pl.BlockSpec((bn, H), row)] * 4 + wspecs,
        out_specs=pl.BlockSpec((bn, H), row),
        out_shape=jax.ShapeDtypeStruct((n, H), jnp.float32),
    )(agg1, agg2, xl1, xl2, *weights)


def kernel(x, feature1, pos_emb, edge_index, batch, params):
    p = params
    n = x.shape[0]
    e = edge_index.shape[1]
    n_pad = ((n + NS * 8 - 1) // (NS * 8)) * (NS * 8)

    eps = e // NS
    porder = jnp.asarray(_edge_perm(e, eps))
    src = edge_index[0][porder]
    dst = edge_index[1][porder]
    ei = jnp.stack([src.reshape(NS, eps), dst.reshape(NS, eps)], axis=1)

    r1 = lambda b: b.reshape(1, -1)

    xl1, xl2 = _run_xlin(x, p['W_lin_1'].T, r1(p['b_lin_1']),
                         p['W_lin_2'].T, r1(p['b_lin_2']))

    f1, f2 = _run_edgefeat(feature1, pos_emb,
                           p['Wf1_1'].T, p['Wf1_2'].T,
                           p['Wf2_1'].T, p['Wf2_2'].T)
    sc_conv = _make_sc_conv(n_pad, e)
    agg1, agg2 = sc_conv(src, dst, ei, xl1, f1, f2)

    wcat0t = p['Wcat0'].T
    weights = [
        p['conv1_Wl'].T, r1(p['conv1_bl']), p['conv1_Wr'].T,
        p['W_lin1'].T, r1(p['b_lin1']),
        p['conv2_Wl'].T, r1(p['conv2_bl']), p['conv2_Wr'].T,
        p['W_lin2'].T, r1(p['b_lin2']),
        wcat0t[:H], wcat0t[H:], r1(p['bcat0']),
        p['Wcat1'].T, r1(p['bcat1']),
        p['Wcat2'].T, r1(p['bcat2']),
        p['Wl0'].T, r1(p['bl0']),
        p['Wl1'].T, r1(p['bl1']),
        p['Wfinal'].T, r1(p['bfinal']),
    ]
    return _run_tail(agg1, agg2, xl1, xl2, weights)


# speedup vs baseline: 3.3097x; 1.0220x over previous
import functools

import numpy as np

import jax
import jax.numpy as jnp
from jax import lax
from jax.experimental import pallas as pl
from jax.experimental.pallas import tpu as pltpu
from jax.experimental.pallas import tpu_sc as plsc

H = 128
L = 16
NC = 2
NS = 16
CH = 128

BE = 8000

def _edge_perm(e, eps):
    half = BE // 2
    u = np.arange(e // 2)
    elo = (u // half) * BE + (u % half)
    ehi = elo + half
    fps = eps // 2
    out = []
    for s in range(NS):
        base = s * fps
        nfull = fps // 64
        for ci in range(nfull):
            rr = base + 64 * ci + np.arange(64)
            out.append(elo[rr])
            out.append(ehi[rr])
        tail = fps - nfull * 64
        if tail:
            rr = base + nfull * 64 + np.arange(tail)
            out.append(elo[rr])
            out.append(ehi[rr])
    return np.concatenate(out)


def _swish(v):
    return v * jax.nn.sigmoid(v)


def _xlin_body(x_ref, w1_ref, b1_ref, w2_ref, b2_ref, o1_ref, o2_ref):
    xb = x_ref[...]
    o1_ref[...] = _swish(
        jnp.dot(xb, w1_ref[...], preferred_element_type=jnp.float32) + b1_ref[...])
    o2_ref[...] = _swish(
        jnp.dot(xb, w2_ref[...], preferred_element_type=jnp.float32) + b2_ref[...])


def _run_xlin(x, w1t, b1, w2t, b2):
    n = x.shape[0]
    bn = 2000
    grid = n // bn
    full = lambda i: (0, 0)
    return pl.pallas_call(
        _xlin_body,
        grid=(grid,),
        in_specs=[
            pl.BlockSpec((bn, H), lambda i: (i, 0)),
            pl.BlockSpec((H, H), full),
            pl.BlockSpec((1, H), full),
            pl.BlockSpec((H, H), full),
            pl.BlockSpec((1, H), full),
        ],
        out_specs=[
            pl.BlockSpec((bn, H), lambda i: (i, 0)),
            pl.BlockSpec((bn, H), lambda i: (i, 0)),
        ],
        out_shape=[
            jax.ShapeDtypeStruct((n, H), jnp.float32),
            jax.ShapeDtypeStruct((n, H), jnp.float32),
        ],
    )(x, w1t, b1, w2t, b2)


def _edgefeat_body(feat_ref, pos_ref, wa1_ref, wa2_ref, wb1_ref, wb2_ref,
                   f1_ref, f2_ref):
    be = feat_ref.shape[0]

    def pack_bf16_pairs(g):
        u = lax.bitcast_convert_type(g, jnp.uint32)
        rnd = (u + jnp.uint32(0x7FFF) + ((u >> 16) & jnp.uint32(1))) >> 16
        return rnd[:be // 2] | (rnd[be // 2:] << 16)

    m1 = jnp.dot(feat_ref[...], wa1_ref[...], preferred_element_type=jnp.float32)
    g1 = jnp.dot(m1, wa2_ref[...], preferred_element_type=jnp.float32)
    f1_ref[...] = pack_bf16_pairs(g1)
    m2 = jnp.dot(pos_ref[...], wb1_ref[...], preferred_element_type=jnp.float32)
    g2 = jnp.dot(m2, wb2_ref[...], preferred_element_type=jnp.float32)
    f2_ref[...] = pack_bf16_pairs(g2)


def _run_edgefeat(feat, pos, wa1t, wa2t, wb1t, wb2t):
    e_pad = feat.shape[0]
    f1d = feat.shape[1]
    ped = pos.shape[1]
    mid = wa1t.shape[1]
    be = BE
    grid = e_pad // be
    full = lambda i: (0, 0)
    return pl.pallas_call(
        _edgefeat_body,
        grid=(grid,),
        in_specs=[
            pl.BlockSpec((be, f1d), lambda i: (i, 0)),
            pl.BlockSpec((be, ped), lambda i: (i, 0)),
            pl.BlockSpec((f1d, mid), full),
            pl.BlockSpec((mid, H), full),
            pl.BlockSpec((ped, mid), full),
            pl.BlockSpec((mid, H), full),
        ],
        out_specs=[
            pl.BlockSpec((be // 2, H), lambda i: (i, 0)),
            pl.BlockSpec((be // 2, H), lambda i: (i, 0)),
        ],
        out_shape=[
            jax.ShapeDtypeStruct((e_pad // 2, H), jnp.uint32),
            jax.ShapeDtypeStruct((e_pad // 2, H), jnp.uint32),
        ],
    )(feat, pos, wa1t, wa2t, wb1t, wb2t)


def _make_sc_conv(n_nodes, e_total):
    eps = e_total // NS
    nchunk = eps // CH
    rem = eps - nchunk * CH
    rows_per_sub = n_nodes // NS
    zchunks = [CH] * (rows_per_sub // CH)
    if rows_per_sub % CH:
        zchunks.append(rows_per_sub % CH)

    mesh = plsc.VectorSubcoreMesh(core_axis_name="c", subcore_axis_name="s")

    @functools.partial(
        pl.kernel,
        out_type=(
            jax.ShapeDtypeStruct((n_nodes, H), jnp.float32),
            jax.ShapeDtypeStruct((n_nodes, H), jnp.float32),
        ),
        mesh=mesh,
        scratch_types=[
            pltpu.VMEM((2, CH), jnp.int32),
            pltpu.VMEM((2, CH), jnp.int32),
            pltpu.VMEM((rem if rem else 8,), jnp.int32),
            pltpu.VMEM((rem if rem else 8,), jnp.int32),
            pltpu.VMEM((CH, H), jnp.float32),
            pltpu.VMEM((CH, H), jnp.float32),
            pltpu.VMEM((CH // 2, H), jnp.uint32),
            pltpu.VMEM((CH // 2, H), jnp.uint32),
            pltpu.VMEM_SHARED((n_nodes, H), jnp.float32),
            pltpu.SemaphoreType.DMA,
            pltpu.SemaphoreType.DMA,
            pltpu.SemaphoreType.DMA,
            pltpu.SemaphoreType.DMA,
            pltpu.SemaphoreType.DMA,
            pltpu.SemaphoreType.DMA,
        ],
    )
    def sc_conv(src_hbm, dst_hbm, ei_hbm, xlin_hbm, f1_hbm, f2_hbm,
                agg1_hbm, agg2_hbm,
                idx0, idx1, sidx_r, didx_r,
                xr0, xr1, fv0, fv1, agg_sp,
                gsem0, gsem1, fsem0, fsem1, ssem0, ssem1):
        c = lax.axis_index("c")
        s = lax.axis_index("s")
        idx = (idx0, idx1)
        xr = (xr0, xr1)
        fv = (fv0, fv1)
        gsem = (gsem0, gsem1)
        fsem = (fsem0, fsem1)
        ssem = (ssem0, ssem1)

        def zrow(r, _):
            for j in range(H // L):
                xr0[r, pl.ds(j * L, L)] = jnp.zeros((L,), jnp.float32)
            return 0
        lax.fori_loop(0, CH, zrow, 0)
        row0 = s * rows_per_sub
        zoff = 0
        for zc in zchunks:
            pltpu.sync_copy(xr0.at[pl.ds(0, zc)],
                            agg_sp.at[pl.ds(row0 + zoff, zc)])
            zoff += zc
        plsc.subcore_barrier()

        def run_conv(f_hbm, out_hbm):
            def issue(ci, b, wait_scatter):
                base2 = pl.multiple_of(s * (eps // 2) + ci * (CH // 2), 8)
                if wait_scatter:
                    pltpu.make_async_copy(xr[b], agg_sp.at[idx[b].at[1]],
                                          ssem[b]).wait()
                pltpu.sync_copy(ei_hbm.at[s, :, pl.ds(ci * CH, CH)], idx[b])
                pltpu.async_copy(xlin_hbm.at[idx[b].at[0]], xr[b], gsem[b])
                pltpu.async_copy(f_hbm.at[pl.ds(base2, CH // 2)], fv[b],
                                 fsem[b])

            def process(ci, b):
                base2 = pl.multiple_of(s * (eps // 2) + ci * (CH // 2), 8)
                pltpu.make_async_copy(xlin_hbm.at[idx[b].at[0]], xr[b],
                                      gsem[b]).wait()
                pltpu.make_async_copy(f_hbm.at[pl.ds(base2, CH // 2)],
                                      fv[b], fsem[b]).wait()

                @plsc.parallel_loop(0, CH // 2, 1, unroll=2)
                def _(q):
                    for j in range(H // L):
                        sl = pl.ds(L * j, L)
                        w = fv[b][q, sl]
                        fa = lax.bitcast_convert_type(w << 16, jnp.float32)
                        fb = lax.bitcast_convert_type(
                            w & jnp.uint32(0xFFFF0000), jnp.float32)
                        xr[b][q, sl] = fa * xr[b][q, sl]
                        hr = CH // 2 + q
                        xr[b][hr, sl] = fb * xr[b][hr, sl]
                pltpu.async_copy(xr[b], agg_sp.at[idx[b].at[1]], ssem[b],
                                 add=True)

            issue(0, 0, False)
            issue(1, 1, False)

            def pair(t, _):
                c0 = 2 * t
                process(c0, 0)
                issue(c0 + 2, 0, True)
                process(c0 + 1, 1)
                issue(c0 + 3, 1, True)
                return 0
            lax.fori_loop(0, nchunk // 2 - 1, pair, 0)
            process(nchunk - 2, 0)
            process(nchunk - 1, 1)
            pltpu.make_async_copy(xr[0], agg_sp.at[idx0.at[1]], ssem0).wait()
            pltpu.make_async_copy(xr[1], agg_sp.at[idx1.at[1]], ssem1).wait()
            if rem:
                base = s * eps + nchunk * CH
                pltpu.sync_copy(src_hbm.at[pl.ds(base, rem)], sidx_r)
                pltpu.sync_copy(dst_hbm.at[pl.ds(base, rem)], didx_r)
                gat = pltpu.async_copy(xlin_hbm.at[sidx_r],
                                       xr0.at[pl.ds(0, rem)], gsem0)
                base2 = pl.multiple_of(
                    s * (eps // 2) + nchunk * (CH // 2), 8)
                pltpu.sync_copy(f_hbm.at[pl.ds(base2, rem // 2)],
                                fv0.at[pl.ds(0, rem // 2)])
                gat.wait()

                @plsc.parallel_loop(0, rem // 2, 1, unroll=2)
                def _(q):
                    for j in range(H // L):
                        sl = pl.ds(L * j, L)
                        w = fv0[q, sl]
                        fa = lax.bitcast_convert_type(w << 16, jnp.float32)
                        fb = lax.bitcast_convert_type(
                            w & jnp.uint32(0xFFFF0000), jnp.float32)
                        xr0[q, sl] = fa * xr0[q, sl]
                        hr = rem // 2 + q
                        xr0[hr, sl] = fb * xr0[hr, sl]
                pltpu.sync_copy(xr0.at[pl.ds(0, rem)],
                                agg_sp.at[didx_r], add=True)
            plsc.subcore_barrier()
            pltpu.sync_copy(agg_sp.at[pl.ds(row0, rows_per_sub)],
                            out_hbm.at[pl.ds(row0, rows_per_sub)])

        @pl.when(c == 0)
        def _():
            run_conv(f1_hbm, agg1_hbm)

        @pl.when(c == 1)
        def _():
            run_conv(f2_hbm, agg2_hbm)

    return sc_conv


def _tail_body(agg1_ref, agg2_ref, xl1_ref, xl2_ref,
               c1wl_ref, c1bl_ref, c1wr_ref, wl1_ref, bl1_ref,
               c2wl_ref, c2bl_ref, c2wr_ref, wl2_ref, bl2_ref,
               wc0a_ref, wc0b_ref, bc0_ref, wc1_ref, bc1_ref, wc2_ref, bc2_ref,
               wl0_ref, bl0_ref, wll1_ref, bll1_ref, wf_ref, bf_ref,
               out_ref):
    f32 = jnp.float32
    xl1 = xl1_ref[...]
    a1 = (jnp.dot(agg1_ref[...], c1wl_ref[...], preferred_element_type=f32)
          + c1bl_ref[...]
          + jnp.dot(xl1, c1wr_ref[...], preferred_element_type=f32))
    h1 = _swish(jnp.dot(a1, wl1_ref[...], preferred_element_type=f32) + bl1_ref[...])
    a2 = (jnp.dot(agg2_ref[...], c2wl_ref[...], preferred_element_type=f32)
          + c2bl_ref[...]
          + jnp.dot(xl1, c2wr_ref[...], preferred_element_type=f32))
    h2 = _swish(jnp.dot(a2, wl2_ref[...], preferred_element_type=f32) + bl2_ref[...])
    h = _swish(jnp.dot(h1, wc0a_ref[...], preferred_element_type=f32)
               + jnp.dot(h2, wc0b_ref[...], preferred_element_type=f32)
               + bc0_ref[...])
    h = _swish(jnp.dot(h, wc1_ref[...], preferred_element_type=f32) + bc1_ref[...])
    h = _swish(jnp.dot(h, wc2_ref[...], preferred_element_type=f32) + bc2_ref[...])
    h = h + xl2_ref[...]
    h = _swish(jnp.dot(h, wl0_ref[...], preferred_element_type=f32) + bl0_ref[...])
    h = _swish(jnp.dot(h, wll1_ref[...], preferred_element_type=f32) + bll1_ref[...])
    out_ref[...] = jnp.dot(h, wf_ref[...], preferred_element_type=f32) + bf_ref[...]


def _run_tail(agg1, agg2, xl1, xl2, weights):
    n = xl1.shape[0]
    bn = 2000
    grid = n // bn
    full = lambda i: (0, 0)
    row = lambda i: (i, 0)
    wspecs = []
    for w in weights:
        wspecs.append(pl.BlockSpec(w.shape, full))
    return pl.pallas_call(
        _tail_body,
        grid=(grid,),
        in_specs=[pl.BlockSpec((bn, H), row)] * 4 + wspecs,
        out_specs=pl.BlockSpec((bn, H), row),
        out_shape=jax.ShapeDtypeStruct((n, H), jnp.float32),
    )(agg1, agg2, xl1, xl2, *weights)


def kernel(x, feature1, pos_emb, edge_index, batch, params):
    p = params
    n = x.shape[0]
    e = edge_index.shape[1]
    n_pad = ((n + NS * 8 - 1) // (NS * 8)) * (NS * 8)

    eps = e // NS
    porder = jnp.asarray(_edge_perm(e, eps))
    src = edge_index[0][porder]
    dst = edge_index[1][porder]
    ei = jnp.stack([src.reshape(NS, eps), dst.reshape(NS, eps)], axis=1)

    r1 = lambda b: b.reshape(1, -1)

    xl1, xl2 = _run_xlin(x, p['W_lin_1'].T, r1(p['b_lin_1']),
                         p['W_lin_2'].T, r1(p['b_lin_2']))

    f1, f2 = _run_edgefeat(feature1, pos_emb,
                           p['Wf1_1'].T, p['Wf1_2'].T,
                           p['Wf2_1'].T, p['Wf2_2'].T)
    sc_conv = _make_sc_conv(n_pad, e)
    agg1, agg2 = sc_conv(src, dst, ei, xl1, f1, f2)

    wcat0t = p['Wcat0'].T
    weights = [
        p['conv1_Wl'].T, r1(p['conv1_bl']), p['conv1_Wr'].T,
        p['W_lin1'].T, r1(p['b_lin1']),
        p['conv2_Wl'].T, r1(p['conv2_bl']), p['conv2_Wr'].T,
        p['W_lin2'].T, r1(p['b_lin2']),
        wcat0t[:H], wcat0t[H:], r1(p['bcat0']),
        p['Wcat1'].T, r1(p['bcat1']),
        p['Wcat2'].T, r1(p['bcat2']),
        p['Wl0'].T, r1(p['bl0']),
        p['Wl1'].T, r1(p['bl1']),
        p['Wfinal'].T, r1(p['bfinal']),
    ]
    return _run_tail(agg1, agg2, xl1, xl2, weights)


# speedup vs baseline: 3.3169x; 1.0022x over previous
import functools

import numpy as np

import jax
import jax.numpy as jnp
from jax import lax
from jax.experimental import pallas as pl
from jax.experimental.pallas import tpu as pltpu
from jax.experimental.pallas import tpu_sc as plsc

H = 128
L = 16
NC = 2
NS = 16
CH = 128

BE = 16000

def _edge_perm(e, eps):
    half = BE // 2
    u = np.arange(e // 2)
    elo = (u // half) * BE + (u % half)
    ehi = elo + half
    fps = eps // 2
    out = []
    for s in range(NS):
        base = s * fps
        nfull = fps // 64
        for ci in range(nfull):
            rr = base + 64 * ci + np.arange(64)
            out.append(elo[rr])
            out.append(ehi[rr])
        tail = fps - nfull * 64
        if tail:
            rr = base + nfull * 64 + np.arange(tail)
            out.append(elo[rr])
            out.append(ehi[rr])
    return np.concatenate(out)


def _swish(v):
    return v * jax.nn.sigmoid(v)


def _xlin_body(x_ref, w1_ref, b1_ref, w2_ref, b2_ref, o1_ref, o2_ref):
    xb = x_ref[...]
    o1_ref[...] = _swish(
        jnp.dot(xb, w1_ref[...], preferred_element_type=jnp.float32) + b1_ref[...])
    o2_ref[...] = _swish(
        jnp.dot(xb, w2_ref[...], preferred_element_type=jnp.float32) + b2_ref[...])


def _run_xlin(x, w1t, b1, w2t, b2):
    n = x.shape[0]
    bn = 2000
    grid = n // bn
    full = lambda i: (0, 0)
    return pl.pallas_call(
        _xlin_body,
        grid=(grid,),
        in_specs=[
            pl.BlockSpec((bn, H), lambda i: (i, 0)),
            pl.BlockSpec((H, H), full),
            pl.BlockSpec((1, H), full),
            pl.BlockSpec((H, H), full),
            pl.BlockSpec((1, H), full),
        ],
        out_specs=[
            pl.BlockSpec((bn, H), lambda i: (i, 0)),
            pl.BlockSpec((bn, H), lambda i: (i, 0)),
        ],
        out_shape=[
            jax.ShapeDtypeStruct((n, H), jnp.float32),
            jax.ShapeDtypeStruct((n, H), jnp.float32),
        ],
    )(x, w1t, b1, w2t, b2)


def _edgefeat_body(feat_ref, pos_ref, wa1_ref, wa2_ref, wb1_ref, wb2_ref,
                   f1_ref, f2_ref):
    be = feat_ref.shape[0]

    def pack_bf16_pairs(g):
        u = lax.bitcast_convert_type(g, jnp.uint32)
        rnd = (u + jnp.uint32(0x7FFF) + ((u >> 16) & jnp.uint32(1))) >> 16
        return rnd[:be // 2] | (rnd[be // 2:] << 16)

    m1 = jnp.dot(feat_ref[...], wa1_ref[...], preferred_element_type=jnp.float32)
    g1 = jnp.dot(m1, wa2_ref[...], preferred_element_type=jnp.float32)
    f1_ref[...] = pack_bf16_pairs(g1)
    m2 = jnp.dot(pos_ref[...], wb1_ref[...], preferred_element_type=jnp.float32)
    g2 = jnp.dot(m2, wb2_ref[...], preferred_element_type=jnp.float32)
    f2_ref[...] = pack_bf16_pairs(g2)


def _run_edgefeat(feat, pos, wa1t, wa2t, wb1t, wb2t):
    e_pad = feat.shape[0]
    f1d = feat.shape[1]
    ped = pos.shape[1]
    mid = wa1t.shape[1]
    be = BE
    grid = e_pad // be
    full = lambda i: (0, 0)
    return pl.pallas_call(
        _edgefeat_body,
        grid=(grid,),
        in_specs=[
            pl.BlockSpec((be, f1d), lambda i: (i, 0)),
            pl.BlockSpec((be, ped), lambda i: (i, 0)),
            pl.BlockSpec((f1d, mid), full),
            pl.BlockSpec((mid, H), full),
            pl.BlockSpec((ped, mid), full),
            pl.BlockSpec((mid, H), full),
        ],
        out_specs=[
            pl.BlockSpec((be // 2, H), lambda i: (i, 0)),
            pl.BlockSpec((be // 2, H), lambda i: (i, 0)),
        ],
        out_shape=[
            jax.ShapeDtypeStruct((e_pad // 2, H), jnp.uint32),
            jax.ShapeDtypeStruct((e_pad // 2, H), jnp.uint32),
        ],
    )(feat, pos, wa1t, wa2t, wb1t, wb2t)


def _make_sc_conv(n_nodes, e_total):
    eps = e_total // NS
    nchunk = eps // CH
    rem = eps - nchunk * CH
    rows_per_sub = n_nodes // NS
    zchunks = [CH] * (rows_per_sub // CH)
    if rows_per_sub % CH:
        zchunks.append(rows_per_sub % CH)

    mesh = plsc.VectorSubcoreMesh(core_axis_name="c", subcore_axis_name="s")

    @functools.partial(
        pl.kernel,
        out_type=(
            jax.ShapeDtypeStruct((n_nodes, H), jnp.float32),
            jax.ShapeDtypeStruct((n_nodes, H), jnp.float32),
        ),
        mesh=mesh,
        scratch_types=[
            pltpu.VMEM((2, CH), jnp.int32),
            pltpu.VMEM((2, CH), jnp.int32),
            pltpu.VMEM((rem if rem else 8,), jnp.int32),
            pltpu.VMEM((rem if rem else 8,), jnp.int32),
            pltpu.VMEM((CH, H), jnp.float32),
            pltpu.VMEM((CH, H), jnp.float32),
            pltpu.VMEM((CH // 2, H), jnp.uint32),
            pltpu.VMEM((CH // 2, H), jnp.uint32),
            pltpu.VMEM_SHARED((n_nodes, H), jnp.float32),
            pltpu.SemaphoreType.DMA,
            pltpu.SemaphoreType.DMA,
            pltpu.SemaphoreType.DMA,
            pltpu.SemaphoreType.DMA,
            pltpu.SemaphoreType.DMA,
            pltpu.SemaphoreType.DMA,
        ],
    )
    def sc_conv(src_hbm, dst_hbm, ei_hbm, xlin_hbm, f1_hbm, f2_hbm,
                agg1_hbm, agg2_hbm,
                idx0, idx1, sidx_r, didx_r,
                xr0, xr1, fv0, fv1, agg_sp,
                gsem0, gsem1, fsem0, fsem1, ssem0, ssem1):
        c = lax.axis_index("c")
        s = lax.axis_index("s")
        idx = (idx0, idx1)
        xr = (xr0, xr1)
        fv = (fv0, fv1)
        gsem = (gsem0, gsem1)
        fsem = (fsem0, fsem1)
        ssem = (ssem0, ssem1)

        def zrow(r, _):
            for j in range(H // L):
                xr0[r, pl.ds(j * L, L)] = jnp.zeros((L,), jnp.float32)
            return 0
        lax.fori_loop(0, CH, zrow, 0)
        row0 = s * rows_per_sub
        zoff = 0
        for zc in zchunks:
            pltpu.sync_copy(xr0.at[pl.ds(0, zc)],
                            agg_sp.at[pl.ds(row0 + zoff, zc)])
            zoff += zc
        plsc.subcore_barrier()

        def run_conv(f_hbm, out_hbm):
            def issue(ci, b, wait_scatter):
                base2 = pl.multiple_of(s * (eps // 2) + ci * (CH // 2), 8)
                if wait_scatter:
                    pltpu.make_async_copy(xr[b], agg_sp.at[idx[b].at[1]],
                                          ssem[b]).wait()
                pltpu.sync_copy(ei_hbm.at[s, :, pl.ds(ci * CH, CH)], idx[b])
                pltpu.async_copy(xlin_hbm.at[idx[b].at[0]], xr[b], gsem[b])
                pltpu.async_copy(f_hbm.at[pl.ds(base2, CH // 2)], fv[b],
                                 fsem[b])

            def process(ci, b):
                base2 = pl.multiple_of(s * (eps // 2) + ci * (CH // 2), 8)
                pltpu.make_async_copy(xlin_hbm.at[idx[b].at[0]], xr[b],
                                      gsem[b]).wait()
                pltpu.make_async_copy(f_hbm.at[pl.ds(base2, CH // 2)],
                                      fv[b], fsem[b]).wait()

                @plsc.parallel_loop(0, CH // 2, 1, unroll=2)
                def _(q):
                    for j in range(H // L):
                        sl = pl.ds(L * j, L)
                        w = fv[b][q, sl]
                        fa = lax.bitcast_convert_type(w << 16, jnp.float32)
                        fb = lax.bitcast_convert_type(
                            w & jnp.uint32(0xFFFF0000), jnp.float32)
                        xr[b][q, sl] = fa * xr[b][q, sl]
                        hr = CH // 2 + q
                        xr[b][hr, sl] = fb * xr[b][hr, sl]
                pltpu.async_copy(xr[b], agg_sp.at[idx[b].at[1]], ssem[b],
                                 add=True)

            issue(0, 0, False)
            issue(1, 1, False)

            def pair(t, _):
                c0 = 2 * t
                process(c0, 0)
                issue(c0 + 2, 0, True)
                process(c0 + 1, 1)
                issue(c0 + 3, 1, True)
                return 0
            lax.fori_loop(0, nchunk // 2 - 1, pair, 0)
            process(nchunk - 2, 0)
            process(nchunk - 1, 1)
            pltpu.make_async_copy(xr[0], agg_sp.at[idx0.at[1]], ssem0).wait()
            pltpu.make_async_copy(xr[1], agg_sp.at[idx1.at[1]], ssem1).wait()
            if rem:
                base = s * eps + nchunk * CH
                pltpu.sync_copy(src_hbm.at[pl.ds(base, rem)], sidx_r)
                pltpu.sync_copy(dst_hbm.at[pl.ds(base, rem)], didx_r)
                gat = pltpu.async_copy(xlin_hbm.at[sidx_r],
                                       xr0.at[pl.ds(0, rem)], gsem0)
                base2 = pl.multiple_of(
                    s * (eps // 2) + nchunk * (CH // 2), 8)
                pltpu.sync_copy(f_hbm.at[pl.ds(base2, rem // 2)],
                                fv0.at[pl.ds(0, rem // 2)])
                gat.wait()

                @plsc.parallel_loop(0, rem // 2, 1, unroll=2)
                def _(q):
                    for j in range(H // L):
                        sl = pl.ds(L * j, L)
                        w = fv0[q, sl]
                        fa = lax.bitcast_convert_type(w << 16, jnp.float32)
                        fb = lax.bitcast_convert_type(
                            w & jnp.uint32(0xFFFF0000), jnp.float32)
                        xr0[q, sl] = fa * xr0[q, sl]
                        hr = rem // 2 + q
                        xr0[hr, sl] = fb * xr0[hr, sl]
                pltpu.sync_copy(xr0.at[pl.ds(0, rem)],
                                agg_sp.at[didx_r], add=True)
            plsc.subcore_barrier()
            pltpu.sync_copy(agg_sp.at[pl.ds(row0, rows_per_sub)],
                            out_hbm.at[pl.ds(row0, rows_per_sub)])

        @pl.when(c == 0)
        def _():
            run_conv(f1_hbm, agg1_hbm)

        @pl.when(c == 1)
        def _():
            run_conv(f2_hbm, agg2_hbm)

    return sc_conv


def _tail_body(agg1_ref, agg2_ref, xl1_ref, xl2_ref,
               c1wl_ref, c1bl_ref, c1wr_ref, wl1_ref, bl1_ref,
               c2wl_ref, c2bl_ref, c2wr_ref, wl2_ref, bl2_ref,
               wc0a_ref, wc0b_ref, bc0_ref, wc1_ref, bc1_ref, wc2_ref, bc2_ref,
               wl0_ref, bl0_ref, wll1_ref, bll1_ref, wf_ref, bf_ref,
               out_ref):
    f32 = jnp.float32
    xl1 = xl1_ref[...]
    a1 = (jnp.dot(agg1_ref[...], c1wl_ref[...], preferred_element_type=f32)
          + c1bl_ref[...]
          + jnp.dot(xl1, c1wr_ref[...], preferred_element_type=f32))
    h1 = _swish(jnp.dot(a1, wl1_ref[...], preferred_element_type=f32) + bl1_ref[...])
    a2 = (jnp.dot(agg2_ref[...], c2wl_ref[...], preferred_element_type=f32)
          + c2bl_ref[...]
          + jnp.dot(xl1, c2wr_ref[...], preferred_element_type=f32))
    h2 = _swish(jnp.dot(a2, wl2_ref[...], preferred_element_type=f32) + bl2_ref[...])
    h = _swish(jnp.dot(h1, wc0a_ref[...], preferred_element_type=f32)
               + jnp.dot(h2, wc0b_ref[...], preferred_element_type=f32)
               + bc0_ref[...])
    h = _swish(jnp.dot(h, wc1_ref[...], preferred_element_type=f32) + bc1_ref[...])
    h = _swish(jnp.dot(h, wc2_ref[...], preferred_element_type=f32) + bc2_ref[...])
    h = h + xl2_ref[...]
    h = _swish(jnp.dot(h, wl0_ref[...], preferred_element_type=f32) + bl0_ref[...])
    h = _swish(jnp.dot(h, wll1_ref[...], preferred_element_type=f32) + bll1_ref[...])
    out_ref[...] = jnp.dot(h, wf_ref[...], preferred_element_type=f32) + bf_ref[...]


def _run_tail(agg1, agg2, xl1, xl2, weights):
    n = xl1.shape[0]
    bn = 2000
    grid = n // bn
    full = lambda i: (0, 0)
    row = lambda i: (i, 0)
    wspecs = []
    for w in weights:
        wspecs.append(pl.BlockSpec(w.shape, full))
    return pl.pallas_call(
        _tail_body,
        grid=(grid,),
        in_specs=[pl.BlockSpec((bn, H), row)] * 4 + wspecs,
        out_specs=pl.BlockSpec((bn, H), row),
        out_shape=jax.ShapeDtypeStruct((n, H), jnp.float32),
    )(agg1, agg2, xl1, xl2, *weights)


def kernel(x, feature1, pos_emb, edge_index, batch, params):
    p = params
    n = x.shape[0]
    e = edge_index.shape[1]
    n_pad = ((n + NS * 8 - 1) // (NS * 8)) * (NS * 8)

    eps = e // NS
    porder = jnp.asarray(_edge_perm(e, eps))
    src = edge_index[0][porder]
    dst = edge_index[1][porder]
    ei = jnp.stack([src.reshape(NS, eps), dst.reshape(NS, eps)], axis=1)

    r1 = lambda b: b.reshape(1, -1)

    xl1, xl2 = _run_xlin(x, p['W_lin_1'].T, r1(p['b_lin_1']),
                         p['W_lin_2'].T, r1(p['b_lin_2']))

    f1, f2 = _run_edgefeat(feature1, pos_emb,
                           p['Wf1_1'].T, p['Wf1_2'].T,
                           p['Wf2_1'].T, p['Wf2_2'].T)
    sc_conv = _make_sc_conv(n_pad, e)
    agg1, agg2 = sc_conv(src, dst, ei, xl1, f1, f2)

    wcat0t = p['Wcat0'].T
    weights = [
        p['conv1_Wl'].T, r1(p['conv1_bl']), p['conv1_Wr'].T,
        p['W_lin1'].T, r1(p['b_lin1']),
        p['conv2_Wl'].T, r1(p['conv2_bl']), p['conv2_Wr'].T,
        p['W_lin2'].T, r1(p['b_lin2']),
        wcat0t[:H], wcat0t[H:], r1(p['bcat0']),
        p['Wcat1'].T, r1(p['bcat1']),
        p['Wcat2'].T, r1(p['bcat2']),
        p['Wl0'].T, r1(p['bl0']),
        p['Wl1'].T, r1(p['bl1']),
        p['Wfinal'].T, r1(p['bfinal']),
    ]
    return _run_tail(agg1, agg2, xl1, xl2, weights)
